# baseline (speedup 1.0000x reference)
import functools

import jax
import jax.numpy as jnp
from jax.experimental import pallas as pl
from jax.experimental.pallas import tpu as pltpu

B = 128
H = 64
W = 64
HID_CNN = 32
HID = 512
NUM_OBJ = 5
EMB = 32

PW = 66
FLAT = PW * PW
HALO = PW + 1
WBUF = FLAT + 2 * HALO
EPS = 1e-5
NPIX = B * H * W


def _tap_offsets():
    return [PW * dy + dx for dy in range(3) for dx in range(3)]


def _conv_taps(w):
    return jnp.transpose(w, (2, 3, 0, 1)).reshape(9, w.shape[0], w.shape[1])


def _stats_rows(acc_masked):
    c = acc_masked.shape[0]
    s1 = jnp.sum(acc_masked, axis=1).reshape(1, c)
    s2 = jnp.sum(acc_masked * acc_masked, axis=1).reshape(1, c)
    pad = 128 - c
    s1 = jnp.pad(s1, ((0, 0), (0, pad)))
    s2 = jnp.pad(s2, ((0, 0), (0, pad)))
    return jnp.concatenate([s1, s2, jnp.zeros((6, 128), jnp.float32)], axis=0)


def _conv_kernel(x_ref, taps_ref, mask_ref, scale_ref, shift_ref,
                 y_ref, st_ref, *, cin, cout, normalize, emit_stats):
    x = x_ref[0]
    if normalize:
        xn = jnp.maximum(x * scale_ref[...] + shift_ref[...], 0.0)
        xn = xn * mask_ref[...]
    else:
        xn = x
    acc = jnp.zeros((cout, FLAT), jnp.float32)
    for t, off in enumerate(_tap_offsets()):
        acc = acc + jnp.dot(taps_ref[t], xn[:, off:off + FLAT],
                            preferred_element_type=jnp.float32)
    inner_mask = mask_ref[0, HALO:HALO + FLAT]
    acc_m = acc * inner_mask
    y_ref[0, :, :HALO] = jnp.zeros((cout, HALO), jnp.float32)
    y_ref[0, :, HALO:HALO + FLAT] = acc
    y_ref[0, :, HALO + FLAT:] = jnp.zeros((cout, HALO), jnp.float32)
    if emit_stats:
        st_ref[0] = _stats_rows(acc_m)


def _conv4_kernel(x_ref, taps_ref, mask_ref, scale_ref, shift_ref, b4_ref,
                  s_ref):
    x = x_ref[0]
    xn = jnp.maximum(x * scale_ref[...] + shift_ref[...], 0.0)
    xn = xn * mask_ref[...]
    acc = jnp.zeros((NUM_OBJ, FLAT), jnp.float32)
    for t, off in enumerate(_tap_offsets()):
        acc = acc + jnp.dot(taps_ref[t], xn[:, off:off + FLAT],
                            preferred_element_type=jnp.float32)
    acc = acc + b4_ref[...]
    s_ref[0] = jax.nn.sigmoid(acc)


def _mlp_kernel(s_ref, w1_ref, b1_ref, w2_ref, b2_ref, lg_ref, lb_ref,
                w3_ref, b3_ref, o_ref):
    h1 = jnp.dot(s_ref[...], w1_ref[...], preferred_element_type=jnp.float32)
    h1 = jnp.maximum(h1 + b1_ref[...], 0.0)
    h2 = jnp.dot(h1, w2_ref[...], preferred_element_type=jnp.float32)
    h2 = h2 + b2_ref[...]
    m = jnp.mean(h2, axis=-1, keepdims=True)
    xm = h2 - m
    v = jnp.mean(xm * xm, axis=-1, keepdims=True)
    h2 = xm * jax.lax.rsqrt(v + EPS) * lg_ref[...] + lb_ref[...]
    h2 = jnp.maximum(h2, 0.0)
    o_ref[...] = jnp.dot(h2, w3_ref[...],
                         preferred_element_type=jnp.float32) + b3_ref[...]


def _conv_call(x, taps, mask, scale, shift, cin, cout, normalize, emit_stats):
    kern = functools.partial(_conv_kernel, cin=cin, cout=cout,
                             normalize=normalize, emit_stats=emit_stats)
    out_shapes = [jax.ShapeDtypeStruct((B, cout, WBUF), jnp.float32),
                  jax.ShapeDtypeStruct((B, 8, 128), jnp.float32)]
    full = lambda b: (0, 0)
    y, st = pl.pallas_call(
        kern,
        grid=(B,),
        in_specs=[
            pl.BlockSpec((1, cin, WBUF), lambda b: (b, 0, 0)),
            pl.BlockSpec((9, cout, cin), lambda b: (0, 0, 0)),
            pl.BlockSpec((1, WBUF), full),
            pl.BlockSpec((cin, 1), full),
            pl.BlockSpec((cin, 1), full),
        ],
        out_specs=[
            pl.BlockSpec((1, cout, WBUF), lambda b: (b, 0, 0)),
            pl.BlockSpec((1, 8, 128), lambda b: (b, 0, 0)),
        ],
        out_shape=out_shapes,
        compiler_params=pltpu.CompilerParams(
            dimension_semantics=("parallel",)),
    )(x, taps, mask, scale, shift)
    return y, st


def _bn_params(st, g, be):
    tot = jnp.sum(st, axis=0)
    c = g.shape[0]
    s1 = tot[0, :c]
    s2 = tot[1, :c]
    mean = s1 / NPIX
    var = s2 / NPIX - mean * mean
    inv = g * jax.lax.rsqrt(var + EPS)
    scale = inv.reshape(c, 1)
    shift = (be - mean * inv).reshape(c, 1)
    return scale, shift


def kernel(obs, W1, b1, g1, be1, W2, b2, g2, be2, W3, b3, g3, be3, W4, b4,
           fc1_w, fc1_b, fc2_w, fc2_b, ln_g, ln_b, fc3_w, fc3_b):
    f32 = jnp.float32
    xp = jnp.pad(obs, ((0, 0), (0, 0), (1, 1), (1, 1))).reshape(B, 3, FLAT)
    xp = jnp.pad(xp, ((0, 0), (0, 5), (HALO, HALO)))
    taps1 = jnp.pad(_conv_taps(W1), ((0, 0), (0, 0), (0, 5)))
    taps2 = _conv_taps(W2)
    taps3 = _conv_taps(W3)
    taps4 = _conv_taps(W4)

    r = jnp.arange(PW)
    inner1d = ((r >= 1) & (r <= H)).astype(f32)
    mask2d = inner1d[:, None] * inner1d[None, :]
    mask = jnp.pad(mask2d.reshape(1, FLAT), ((0, 0), (HALO, HALO)))

    zero8 = jnp.zeros((8, 1), f32)
    one8 = jnp.ones((8, 1), f32)
    y1, st1 = _conv_call(xp, taps1, mask, one8, zero8, 8, HID_CNN,
                         normalize=False, emit_stats=True)
    sc1, sh1 = _bn_params(st1, g1, be1)
    y2, st2 = _conv_call(y1, taps2, mask, sc1, sh1, HID_CNN, HID_CNN,
                         normalize=True, emit_stats=True)
    sc2, sh2 = _bn_params(st2, g2, be2)
    y3, st3 = _conv_call(y2, taps3, mask, sc2, sh2, HID_CNN, HID_CNN,
                         normalize=True, emit_stats=True)
    sc3, sh3 = _bn_params(st3, g3, be3)

    s = pl.pallas_call(
        _conv4_kernel,
        grid=(B,),
        in_specs=[
            pl.BlockSpec((1, HID_CNN, WBUF), lambda b: (b, 0, 0)),
            pl.BlockSpec((9, NUM_OBJ, HID_CNN), lambda b: (0, 0, 0)),
            pl.BlockSpec((1, WBUF), lambda b: (0, 0)),
            pl.BlockSpec((HID_CNN, 1), lambda b: (0, 0)),
            pl.BlockSpec((HID_CNN, 1), lambda b: (0, 0)),
            pl.BlockSpec((NUM_OBJ, 1), lambda b: (0, 0)),
        ],
        out_specs=pl.BlockSpec((1, NUM_OBJ, FLAT), lambda b: (b, 0, 0)),
        out_shape=jax.ShapeDtypeStruct((B, NUM_OBJ, FLAT), f32),
        compiler_params=pltpu.CompilerParams(
            dimension_semantics=("parallel",)),
    )(y3, taps4, mask, sc3, sh3, b4.reshape(NUM_OBJ, 1))

    fc1e = jnp.pad(fc1_w.reshape(HID, H, W), ((0, 0), (1, 1), (1, 1)))
    fc1e = fc1e.reshape(HID, FLAT)
    srows = s.reshape(B * NUM_OBJ, FLAT)

    out = pl.pallas_call(
        _mlp_kernel,
        grid=(5,),
        in_specs=[
            pl.BlockSpec((128, FLAT), lambda i: (i, 0)),
            pl.BlockSpec((FLAT, HID), lambda i: (0, 0)),
            pl.BlockSpec((1, HID), lambda i: (0, 0)),
            pl.BlockSpec((HID, HID), lambda i: (0, 0)),
            pl.BlockSpec((1, HID), lambda i: (0, 0)),
            pl.BlockSpec((1, HID), lambda i: (0, 0)),
            pl.BlockSpec((1, HID), lambda i: (0, 0)),
            pl.BlockSpec((HID, EMB), lambda i: (0, 0)),
            pl.BlockSpec((1, EMB), lambda i: (0, 0)),
        ],
        out_specs=pl.BlockSpec((128, EMB), lambda i: (i, 0)),
        out_shape=jax.ShapeDtypeStruct((B * NUM_OBJ, EMB), f32),
        compiler_params=pltpu.CompilerParams(
            dimension_semantics=("parallel",)),
    )(srows, fc1e.T, fc1_b.reshape(1, HID), fc2_w.T, fc2_b.reshape(1, HID),
      ln_g.reshape(1, HID), ln_b.reshape(1, HID), fc3_w.T,
      fc3_b.reshape(1, EMB))

    return out.reshape(B, NUM_OBJ, EMB)


# speedup vs baseline: 1.3949x; 1.3949x over previous
import functools

import jax
import jax.numpy as jnp
from jax.experimental import pallas as pl
from jax.experimental.pallas import tpu as pltpu

B = 128
H = 64
W = 64
HID_CNN = 32
HID = 512
NUM_OBJ = 5
EMB = 32

PW = 66
FLAT = PW * PW
HALO = PW + 1
WBUF = FLAT + 2 * HALO
XCW = FLAT + 2 * PW
EPS = 1e-5
NPIX = B * H * W
IMGS = 2

f32 = jnp.float32
bf16 = jnp.bfloat16


def _conv_wg(w):
    return jnp.transpose(w, (2, 0, 3, 1)).reshape(3, w.shape[0],
                                                  3 * w.shape[1])


def _conv3x3(xn, wg, cout):
    xc = jnp.concatenate([xn[:, 0:XCW], xn[:, 1:1 + XCW], xn[:, 2:2 + XCW]],
                         axis=0)
    acc = jnp.zeros((cout, FLAT), f32)
    for dy in range(3):
        acc = acc + jnp.dot(wg[dy], xc[:, PW * dy:PW * dy + FLAT],
                            preferred_element_type=f32)
    return acc


def _stats_rows(parts, mask):
    c = parts[0].shape[0]
    s1 = jnp.zeros((c,), f32)
    s2 = jnp.zeros((c,), f32)
    for acc in parts:
        am = acc * mask
        s1 = s1 + jnp.sum(am, axis=1)
        s2 = s2 + jnp.sum(am * am, axis=1)
    pad = 128 - c
    s1 = jnp.pad(s1.reshape(1, c), ((0, 0), (0, pad)))
    s2 = jnp.pad(s2.reshape(1, c), ((0, 0), (0, pad)))
    return jnp.concatenate([s1, s2, jnp.zeros((6, 128), f32)], axis=0)


def _conv_kernel(x_ref, wg_ref, scale_ref, shift_ref, mask_ref,
                 y_ref, st_ref, *, cout, normalize, emit_stats):
    parts = []
    for i in range(IMGS):
        x = x_ref[i]
        if normalize:
            xn = jnp.maximum(x * scale_ref[...] + shift_ref[...], 0.0)
            xn = xn.astype(bf16)
        else:
            xn = x
        acc = _conv3x3(xn, wg_ref, cout)
        y_ref[i, :, :HALO] = jnp.zeros((cout, HALO), bf16)
        y_ref[i, :, HALO:HALO + FLAT] = acc.astype(bf16)
        y_ref[i, :, HALO + FLAT:] = jnp.zeros((cout, HALO), bf16)
        parts.append(acc)
    if emit_stats:
        st_ref[0] = _stats_rows(parts, mask_ref[0, HALO:HALO + FLAT])


def _conv4_kernel(x_ref, wg_ref, scale_ref, shift_ref, b4_ref, s_ref):
    for i in range(IMGS):
        xn = jnp.maximum(x_ref[i] * scale_ref[...] + shift_ref[...], 0.0)
        acc = _conv3x3(xn.astype(bf16), wg_ref, NUM_OBJ)
        s_ref[i] = jax.nn.sigmoid(acc + b4_ref[...]).astype(bf16)


def _mlp_kernel(s_ref, w1_ref, b1_ref, w2_ref, b2_ref, lg_ref, lb_ref,
                w3_ref, b3_ref, o_ref):
    h1 = jnp.dot(s_ref[...], w1_ref[...], preferred_element_type=f32)
    h1 = jnp.maximum(h1 + b1_ref[...], 0.0).astype(bf16)
    h2 = jnp.dot(h1, w2_ref[...], preferred_element_type=f32)
    h2 = h2 + b2_ref[...]
    m = jnp.mean(h2, axis=-1, keepdims=True)
    xm = h2 - m
    v = jnp.mean(xm * xm, axis=-1, keepdims=True)
    h2 = xm * jax.lax.rsqrt(v + EPS) * lg_ref[...] + lb_ref[...]
    h2 = jnp.maximum(h2, 0.0).astype(bf16)
    o_ref[...] = jnp.dot(h2, w3_ref[...],
                         preferred_element_type=f32) + b3_ref[...]


def _conv_call(x, wg, scale, shift, mask, cin, cout, normalize, emit_stats):
    kern = functools.partial(_conv_kernel, cout=cout,
                             normalize=normalize, emit_stats=emit_stats)
    nblk = B // IMGS
    full = lambda b: (0, 0)
    y, st = pl.pallas_call(
        kern,
        grid=(nblk,),
        in_specs=[
            pl.BlockSpec((IMGS, cin, WBUF), lambda b: (b, 0, 0)),
            pl.BlockSpec((3, cout, 3 * cin), lambda b: (0, 0, 0)),
            pl.BlockSpec((cin, WBUF), full),
            pl.BlockSpec((cin, WBUF), full),
            pl.BlockSpec((1, WBUF), full),
        ],
        out_specs=[
            pl.BlockSpec((IMGS, cout, WBUF), lambda b: (b, 0, 0)),
            pl.BlockSpec((1, 8, 128), lambda b: (b, 0, 0)),
        ],
        out_shape=[jax.ShapeDtypeStruct((B, cout, WBUF), bf16),
                   jax.ShapeDtypeStruct((nblk, 8, 128), f32)],
        compiler_params=pltpu.CompilerParams(
            dimension_semantics=("parallel",)),
    )(x, wg, scale, shift, mask)
    return y, st


def _bn_planes(st, g, be, maskful):
    tot = jnp.sum(st, axis=0)
    c = g.shape[0]
    s1 = tot[0, :c]
    s2 = tot[1, :c]
    mean = s1 / NPIX
    var = s2 / NPIX - mean * mean
    inv = g * jax.lax.rsqrt(var + EPS)
    scale = inv.reshape(c, 1) * maskful
    shift = (be - mean * inv).reshape(c, 1) * maskful
    return scale, shift


def kernel(obs, W1, b1, g1, be1, W2, b2, g2, be2, W3, b3, g3, be3, W4, b4,
           fc1_w, fc1_b, fc2_w, fc2_b, ln_g, ln_b, fc3_w, fc3_b):
    xp = jnp.pad(obs, ((0, 0), (0, 0), (1, 1), (1, 1))).reshape(B, 3, FLAT)
    xp = jnp.pad(xp, ((0, 0), (0, 5), (HALO, HALO))).astype(bf16)
    wg1 = _conv_wg(jnp.pad(W1, ((0, 0), (0, 5), (0, 0), (0, 0)))).astype(bf16)
    wg2 = _conv_wg(W2).astype(bf16)
    wg3 = _conv_wg(W3).astype(bf16)
    wg4 = _conv_wg(W4).astype(bf16)

    r = jnp.arange(PW)
    inner1d = ((r >= 1) & (r <= H)).astype(f32)
    mask2d = inner1d[:, None] * inner1d[None, :]
    maskful = jnp.pad(mask2d.reshape(1, FLAT), ((0, 0), (HALO, HALO)))

    one_sc = jnp.ones((8, WBUF), f32)
    zero_sc = jnp.zeros((8, WBUF), f32)
    y1, st1 = _conv_call(xp, wg1, one_sc, zero_sc, maskful, 8, HID_CNN,
                         normalize=False, emit_stats=True)
    sc1, sh1 = _bn_planes(st1, g1, be1, maskful)
    y2, st2 = _conv_call(y1, wg2, sc1, sh1, maskful, HID_CNN, HID_CNN,
                         normalize=True, emit_stats=True)
    sc2, sh2 = _bn_planes(st2, g2, be2, maskful)
    y3, st3 = _conv_call(y2, wg3, sc2, sh2, maskful, HID_CNN, HID_CNN,
                         normalize=True, emit_stats=True)
    sc3, sh3 = _bn_planes(st3, g3, be3, maskful)

    s = pl.pallas_call(
        _conv4_kernel,
        grid=(B // IMGS,),
        in_specs=[
            pl.BlockSpec((IMGS, HID_CNN, WBUF), lambda b: (b, 0, 0)),
            pl.BlockSpec((3, NUM_OBJ, 3 * HID_CNN), lambda b: (0, 0, 0)),
            pl.BlockSpec((HID_CNN, WBUF), lambda b: (0, 0)),
            pl.BlockSpec((HID_CNN, WBUF), lambda b: (0, 0)),
            pl.BlockSpec((NUM_OBJ, 1), lambda b: (0, 0)),
        ],
        out_specs=pl.BlockSpec((IMGS, NUM_OBJ, FLAT), lambda b: (b, 0, 0)),
        out_shape=jax.ShapeDtypeStruct((B, NUM_OBJ, FLAT), bf16),
        compiler_params=pltpu.CompilerParams(
            dimension_semantics=("parallel",)),
    )(y3, wg4, sc3, sh3, b4.reshape(NUM_OBJ, 1))

    fc1e = jnp.pad(fc1_w.reshape(HID, H, W), ((0, 0), (1, 1), (1, 1)))
    fc1e = fc1e.reshape(HID, FLAT)
    srows = s.reshape(B * NUM_OBJ, FLAT)

    out = pl.pallas_call(
        _mlp_kernel,
        grid=(5,),
        in_specs=[
            pl.BlockSpec((128, FLAT), lambda i: (i, 0)),
            pl.BlockSpec((FLAT, HID), lambda i: (0, 0)),
            pl.BlockSpec((1, HID), lambda i: (0, 0)),
            pl.BlockSpec((HID, HID), lambda i: (0, 0)),
            pl.BlockSpec((1, HID), lambda i: (0, 0)),
            pl.BlockSpec((1, HID), lambda i: (0, 0)),
            pl.BlockSpec((1, HID), lambda i: (0, 0)),
            pl.BlockSpec((HID, EMB), lambda i: (0, 0)),
            pl.BlockSpec((1, EMB), lambda i: (0, 0)),
        ],
        out_specs=pl.BlockSpec((128, EMB), lambda i: (i, 0)),
        out_shape=jax.ShapeDtypeStruct((B * NUM_OBJ, EMB), f32),
        compiler_params=pltpu.CompilerParams(
            dimension_semantics=("parallel",)),
    )(srows, fc1e.T.astype(bf16), fc1_b.reshape(1, HID),
      fc2_w.T.astype(bf16), fc2_b.reshape(1, HID),
      ln_g.reshape(1, HID), ln_b.reshape(1, HID), fc3_w.T.astype(bf16),
      fc3_b.reshape(1, EMB))

    return out.reshape(B, NUM_OBJ, EMB)


# speedup vs baseline: 1.6195x; 1.1611x over previous
import functools

import jax
import jax.numpy as jnp
from jax.experimental import pallas as pl
from jax.experimental.pallas import tpu as pltpu

B = 128
H = 64
W = 64
HID_CNN = 32
HID = 512
NUM_OBJ = 5
EMB = 32

PW = 66
FLAT = PW * PW
HALO = PW + 1
WBUF = FLAT + 2 * HALO
XCW = FLAT + 2 * PW
EPS = 1e-5
NPIX = B * H * W
IMGS = 4
TW = FLAT
TILES = [(j * TW, min(TW, FLAT - j * TW))
         for j in range((FLAT + TW - 1) // TW)]
NST = 2 * ((len(TILES) + 3) // 4) * 4

f32 = jnp.float32
bf16 = jnp.bfloat16


def _conv_wg(w):
    return jnp.transpose(w, (2, 0, 3, 1)).reshape(3, w.shape[0],
                                                  3 * w.shape[1])


def _tile_conv(x_ref, i, scale_ref, shift_ref, wg_ref, a, tw, cout,
               normalize):
    xt = x_ref[i, :, a:a + tw + 2 * HALO]
    if normalize:
        xt = jnp.maximum(xt * scale_ref[:, a:a + tw + 2 * HALO]
                         + shift_ref[:, a:a + tw + 2 * HALO],
                         jnp.zeros((), bf16))
    xc = jnp.concatenate([xt[:, 0:tw + 2 * PW], xt[:, 1:1 + tw + 2 * PW],
                          xt[:, 2:2 + tw + 2 * PW]], axis=0)
    acc = jnp.zeros((cout, tw), f32)
    for dy in range(3):
        acc = acc + jnp.dot(wg_ref[dy], xc[:, PW * dy:PW * dy + tw],
                            preferred_element_type=f32)
    return acc


def _conv_kernel(x_ref, wg_ref, scale_ref, shift_ref, mask_ref,
                 y_ref, st_ref, *, cout, normalize, emit_stats):
    pad = 128 - cout
    for i in range(IMGS):
        y_ref[i, :, :HALO] = jnp.zeros((cout, HALO), bf16)
        y_ref[i, :, HALO + FLAT:] = jnp.zeros((cout, HALO), bf16)
    for j, (a, tw) in enumerate(TILES):
        accs = [_tile_conv(x_ref, i, scale_ref, shift_ref, wg_ref, a, tw,
                           cout, normalize) for i in range(IMGS)]
        for i, acc in enumerate(accs):
            y_ref[i, :, HALO + a:HALO + a + tw] = acc.astype(bf16)
        if emit_stats:
            m = mask_ref[0, HALO + a:HALO + a + tw]
            s1 = jnp.zeros((1, cout), f32)
            s2 = jnp.zeros((1, cout), f32)
            for acc in accs:
                am = acc * m
                s1 = s1 + jnp.sum(am, axis=1).reshape(1, cout)
                s2 = s2 + jnp.sum(am * am, axis=1).reshape(1, cout)
            st_ref[0, 2 * j:2 * j + 2] = jnp.pad(
                jnp.concatenate([s1, s2], axis=0), ((0, 0), (0, pad)))
    if emit_stats:
        for r in range(2 * len(TILES), NST):
            st_ref[0, r:r + 1] = jnp.zeros((1, 128), f32)


def _conv4_kernel(x_ref, wg_ref, scale_ref, shift_ref, b4_ref, s_ref):
    for a, tw in TILES:
        for i in range(IMGS):
            acc = _tile_conv(x_ref, i, scale_ref, shift_ref, wg_ref, a, tw,
                             NUM_OBJ, True)
            s_ref[i, :, a:a + tw] = jax.nn.sigmoid(
                acc + b4_ref[...]).astype(bf16)


def _mlp_kernel(s_ref, w1_ref, b1_ref, w2_ref, b2_ref, lg_ref, lb_ref,
                w3_ref, b3_ref, o_ref):
    h1 = jnp.dot(s_ref[...], w1_ref[...], preferred_element_type=f32)
    h1 = jnp.maximum(h1 + b1_ref[...], 0.0).astype(bf16)
    h2 = jnp.dot(h1, w2_ref[...], preferred_element_type=f32)
    h2 = h2 + b2_ref[...]
    m = jnp.mean(h2, axis=-1, keepdims=True)
    xm = h2 - m
    v = jnp.mean(xm * xm, axis=-1, keepdims=True)
    h2 = xm * jax.lax.rsqrt(v + EPS) * lg_ref[...] + lb_ref[...]
    h2 = jnp.maximum(h2, 0.0).astype(bf16)
    o_ref[...] = jnp.dot(h2, w3_ref[...],
                         preferred_element_type=f32) + b3_ref[...]


def _conv_call(x, wg, scale, shift, mask, cin, cout, normalize, emit_stats):
    kern = functools.partial(_conv_kernel, cout=cout,
                             normalize=normalize, emit_stats=emit_stats)
    nblk = B // IMGS
    full = lambda b: (0, 0)
    y, st = pl.pallas_call(
        kern,
        grid=(nblk,),
        in_specs=[
            pl.BlockSpec((IMGS, cin, WBUF), lambda b: (b, 0, 0)),
            pl.BlockSpec((3, cout, 3 * cin), lambda b: (0, 0, 0)),
            pl.BlockSpec((cin, WBUF), full),
            pl.BlockSpec((cin, WBUF), full),
            pl.BlockSpec((1, WBUF), full),
        ],
        out_specs=[
            pl.BlockSpec((IMGS, cout, WBUF), lambda b: (b, 0, 0)),
            pl.BlockSpec((1, NST, 128), lambda b: (b, 0, 0)),
        ],
        out_shape=[jax.ShapeDtypeStruct((B, cout, WBUF), bf16),
                   jax.ShapeDtypeStruct((nblk, NST, 128), f32)],
        compiler_params=pltpu.CompilerParams(
            dimension_semantics=("parallel",)),
    )(x, wg, scale, shift, mask)
    return y, st


def _bn_planes(st, g, be, maskful):
    tot = jnp.sum(st.reshape(-1, 2, 128), axis=0)
    c = g.shape[0]
    s1 = tot[0, :c]
    s2 = tot[1, :c]
    mean = s1 / NPIX
    var = s2 / NPIX - mean * mean
    inv = g * jax.lax.rsqrt(var + EPS)
    scale = (inv.reshape(c, 1) * maskful).astype(bf16)
    shift = ((be - mean * inv).reshape(c, 1) * maskful).astype(bf16)
    return scale, shift


def kernel(obs, W1, b1, g1, be1, W2, b2, g2, be2, W3, b3, g3, be3, W4, b4,
           fc1_w, fc1_b, fc2_w, fc2_b, ln_g, ln_b, fc3_w, fc3_b):
    xp = jnp.pad(obs, ((0, 0), (0, 0), (1, 1), (1, 1))).reshape(B, 3, FLAT)
    xp = jnp.pad(xp, ((0, 0), (0, 5), (HALO, HALO))).astype(bf16)
    wg1 = _conv_wg(jnp.pad(W1, ((0, 0), (0, 5), (0, 0), (0, 0)))).astype(bf16)
    wg2 = _conv_wg(W2).astype(bf16)
    wg3 = _conv_wg(W3).astype(bf16)
    wg4 = _conv_wg(W4).astype(bf16)

    r = jnp.arange(PW)
    inner1d = ((r >= 1) & (r <= H)).astype(f32)
    mask2d = inner1d[:, None] * inner1d[None, :]
    maskful = jnp.pad(mask2d.reshape(1, FLAT), ((0, 0), (HALO, HALO)))

    one_sc = jnp.ones((8, WBUF), bf16)
    zero_sc = jnp.zeros((8, WBUF), bf16)
    y1, st1 = _conv_call(xp, wg1, one_sc, zero_sc, maskful, 8, HID_CNN,
                         normalize=False, emit_stats=True)
    sc1, sh1 = _bn_planes(st1, g1, be1, maskful)
    y2, st2 = _conv_call(y1, wg2, sc1, sh1, maskful, HID_CNN, HID_CNN,
                         normalize=True, emit_stats=True)
    sc2, sh2 = _bn_planes(st2, g2, be2, maskful)
    y3, st3 = _conv_call(y2, wg3, sc2, sh2, maskful, HID_CNN, HID_CNN,
                         normalize=True, emit_stats=True)
    sc3, sh3 = _bn_planes(st3, g3, be3, maskful)

    s = pl.pallas_call(
        _conv4_kernel,
        grid=(B // IMGS,),
        in_specs=[
            pl.BlockSpec((IMGS, HID_CNN, WBUF), lambda b: (b, 0, 0)),
            pl.BlockSpec((3, NUM_OBJ, 3 * HID_CNN), lambda b: (0, 0, 0)),
            pl.BlockSpec((HID_CNN, WBUF), lambda b: (0, 0)),
            pl.BlockSpec((HID_CNN, WBUF), lambda b: (0, 0)),
            pl.BlockSpec((NUM_OBJ, 1), lambda b: (0, 0)),
        ],
        out_specs=pl.BlockSpec((IMGS, NUM_OBJ, FLAT), lambda b: (b, 0, 0)),
        out_shape=jax.ShapeDtypeStruct((B, NUM_OBJ, FLAT), bf16),
        compiler_params=pltpu.CompilerParams(
            dimension_semantics=("parallel",)),
    )(y3, wg4, sc3, sh3, b4.reshape(NUM_OBJ, 1))

    fc1e = jnp.pad(fc1_w.reshape(HID, H, W), ((0, 0), (1, 1), (1, 1)))
    fc1e = fc1e.reshape(HID, FLAT)
    srows = s.reshape(B * NUM_OBJ, FLAT)

    out = pl.pallas_call(
        _mlp_kernel,
        grid=(5,),
        in_specs=[
            pl.BlockSpec((128, FLAT), lambda i: (i, 0)),
            pl.BlockSpec((FLAT, HID), lambda i: (0, 0)),
            pl.BlockSpec((1, HID), lambda i: (0, 0)),
            pl.BlockSpec((HID, HID), lambda i: (0, 0)),
            pl.BlockSpec((1, HID), lambda i: (0, 0)),
            pl.BlockSpec((1, HID), lambda i: (0, 0)),
            pl.BlockSpec((1, HID), lambda i: (0, 0)),
            pl.BlockSpec((HID, EMB), lambda i: (0, 0)),
            pl.BlockSpec((1, EMB), lambda i: (0, 0)),
        ],
        out_specs=pl.BlockSpec((128, EMB), lambda i: (i, 0)),
        out_shape=jax.ShapeDtypeStruct((B * NUM_OBJ, EMB), f32),
        compiler_params=pltpu.CompilerParams(
            dimension_semantics=("parallel",)),
    )(srows, fc1e.T.astype(bf16), fc1_b.reshape(1, HID),
      fc2_w.T.astype(bf16), fc2_b.reshape(1, HID),
      ln_g.reshape(1, HID), ln_b.reshape(1, HID), fc3_w.T.astype(bf16),
      fc3_b.reshape(1, EMB))

    return out.reshape(B, NUM_OBJ, EMB)
